# baseline (speedup 1.0000x reference)
import functools

import jax
import jax.numpy as jnp
from jax import lax
from jax.experimental import pallas as pl
from jax.experimental.pallas import tpu as pltpu
from jax.experimental.pallas import tpu_sc as plsc

_NC = 2
_NS = 16
_NW = _NC * _NS


def _sc_gather_packed(flat_ids, word_table, chunk_rows):
    n = flat_ids.shape[0]
    e = word_table.shape[1]
    n2 = n // 2
    per_w = n2 // _NW
    n_chunks = per_w // chunk_rows

    mesh = plsc.VectorSubcoreMesh(
        core_axis_name="c", subcore_axis_name="s", num_cores=_NC, num_subcores=_NS
    )

    @functools.partial(
        pl.kernel,
        out_type=jax.ShapeDtypeStruct((n2, 2 * e), jnp.float32),
        mesh=mesh,
        scratch_types=[
            pltpu.VMEM((chunk_rows,), jnp.int32),
            pltpu.VMEM((chunk_rows,), jnp.int32),
            pltpu.VMEM((chunk_rows, e), jnp.float32),
            pltpu.VMEM((chunk_rows, e), jnp.float32),
            pltpu.SemaphoreType.DMA,
        ],
        compiler_params=pltpu.CompilerParams(use_tc_tiling_on_sc=False),
    )
    def gather_kernel(ids_hbm, table_hbm, out_hbm, idx_l, idx_r, lv, rv, sem):
        wid = lax.axis_index("s") * _NC + lax.axis_index("c")
        base = wid * per_w

        def body(i, carry):
            off = pl.multiple_of(base + i * chunk_rows, 8)
            pltpu.sync_copy(ids_hbm.at[pl.ds(off, chunk_rows)], idx_l)
            pltpu.sync_copy(ids_hbm.at[pl.ds(n2 + off, chunk_rows)], idx_r)
            cl = pltpu.async_copy(table_hbm.at[idx_l], lv, sem)
            cr = pltpu.async_copy(table_hbm.at[idx_r], rv, sem)
            cl.wait()
            cr.wait()
            pltpu.sync_copy(lv, out_hbm.at[pl.ds(off, chunk_rows), pl.ds(0, e)])
            pltpu.sync_copy(rv, out_hbm.at[pl.ds(off, chunk_rows), pl.ds(e, e)])
            return carry

        lax.fori_loop(0, n_chunks, body, 0)

    return gather_kernel(flat_ids, word_table)


def _tc_add_ln(xp, tt3, pos_table, type_table, gamma, beta, bb):
    b, l, _ = tt3.shape
    e = pos_table.shape[1]
    b2 = b // 2
    hb = b2 // bb
    rb = bb * l

    def body(x_ref, tt_ref, pos_ref, type_ref, g_ref, b_ref, o_ref):
        h = pl.program_id(0)
        x = x_ref[...]
        pos = pos_ref[...]
        t0 = type_ref[0, :]
        dt = type_ref[1, :] - t0
        g = g_ref[...]
        bb_ = b_ref[...]
        jmat = jnp.full((e, e), 1.0 / e, dtype=jnp.float32)

        def do(xh):
            xh = xh.reshape(bb, l, e)
            tt = tt_ref[...]
            emb = xh + pos[None, :, :] + t0[None, None, :] + tt * dt[None, None, :]
            mean = jax.lax.dot_general(
                emb, jmat, (((2,), (0,)), ((), ())),
                preferred_element_type=jnp.float32)
            c = emb - mean
            var = jax.lax.dot_general(
                c * c, jmat, (((2,), (0,)), ((), ())),
                preferred_element_type=jnp.float32)
            inv = lax.rsqrt(var + 1e-5)
            o_ref[...] = c * inv * g + bb_

        @pl.when(h == 0)
        def _():
            do(x[:, 0:e])

        @pl.when(h == 1)
        def _():
            do(x[:, e:2 * e])

    return pl.pallas_call(
        body,
        grid=(2, hb),
        in_specs=[
            pl.BlockSpec((rb, 2 * e), lambda h, i: (i, 0)),
            pl.BlockSpec((bb, l, 1), lambda h, i: (h * hb + i, 0, 0)),
            pl.BlockSpec((l, e), lambda h, i: (0, 0)),
            pl.BlockSpec((2, e), lambda h, i: (0, 0)),
            pl.BlockSpec((1, e), lambda h, i: (0, 0)),
            pl.BlockSpec((1, e), lambda h, i: (0, 0)),
        ],
        out_specs=pl.BlockSpec((bb, l, e), lambda h, i: (h * hb + i, 0, 0)),
        out_shape=jax.ShapeDtypeStruct((b, l, e), jnp.float32),
    )(xp, tt3, pos_table, type_table, gamma.reshape(1, e), beta.reshape(1, e))


def kernel(input_ids, token_type_ids, word_table, pos_table, type_table, ln_gamma, ln_beta):
    b, l = input_ids.shape
    flat_ids = input_ids.reshape(b * l)
    xp = _sc_gather_packed(flat_ids, word_table, chunk_rows=256)
    tt3 = token_type_ids.astype(jnp.float32).reshape(b, l, 1)
    return _tc_add_ln(xp, tt3, pos_table[:l], type_table, ln_gamma, ln_beta, bb=8)


# speedup vs baseline: 1.2917x; 1.2917x over previous
import functools

import jax
import jax.numpy as jnp
from jax import lax
from jax.experimental import pallas as pl
from jax.experimental.pallas import tpu as pltpu
from jax.experimental.pallas import tpu_sc as plsc

_NC = 2
_NS = 16
_NW = _NC * _NS


def _sc_gather_packed(flat_ids, table, chunk_rows):
    n = flat_ids.shape[0]
    e = table.shape[1]
    n2 = n // 2
    per_w = n2 // _NW
    n_chunks = per_w // chunk_rows

    mesh = plsc.VectorSubcoreMesh(
        core_axis_name="c", subcore_axis_name="s", num_cores=_NC, num_subcores=_NS
    )

    @functools.partial(
        pl.kernel,
        out_type=jax.ShapeDtypeStruct((n2, 2 * e), jnp.float32),
        mesh=mesh,
        scratch_types=[
            pltpu.VMEM((chunk_rows,), jnp.int32),
            pltpu.VMEM((chunk_rows,), jnp.int32),
            pltpu.VMEM((chunk_rows, e), jnp.float32),
            pltpu.VMEM((chunk_rows, e), jnp.float32),
            pltpu.SemaphoreType.DMA,
        ],
        compiler_params=pltpu.CompilerParams(use_tc_tiling_on_sc=False),
    )
    def gather_kernel(ids_hbm, table_hbm, out_hbm, idx_l, idx_r, lv, rv, sem):
        wid = lax.axis_index("s") * _NC + lax.axis_index("c")
        base = wid * per_w

        def body(i, carry):
            off = pl.multiple_of(base + i * chunk_rows, 8)
            pltpu.sync_copy(ids_hbm.at[pl.ds(off, chunk_rows)], idx_l)
            pltpu.sync_copy(ids_hbm.at[pl.ds(n2 + off, chunk_rows)], idx_r)
            cl = pltpu.async_copy(table_hbm.at[idx_l], lv, sem)
            cr = pltpu.async_copy(table_hbm.at[idx_r], rv, sem)
            cl.wait()
            cr.wait()
            pltpu.sync_copy(lv, out_hbm.at[pl.ds(off, chunk_rows), pl.ds(0, e)])
            pltpu.sync_copy(rv, out_hbm.at[pl.ds(off, chunk_rows), pl.ds(e, e)])
            return carry

        lax.fori_loop(0, n_chunks, body, 0)

    return gather_kernel(flat_ids, table)


def _tc_add_ln(xp, pos_tiled, gamma, beta, b, l, e, bb):
    b2 = b // 2
    rb = bb * l

    def body(x_ref, pos_ref, g_ref, b_ref, o_ref):
        x = x_ref[...] + pos_ref[...]
        g = g_ref[...]
        bt = b_ref[...]
        for h in range(2):
            xh = x[:, h * e:(h + 1) * e]
            mean = jnp.mean(xh, axis=-1, keepdims=True)
            c = xh - mean
            var = jnp.mean(c * c, axis=-1, keepdims=True)
            inv = lax.rsqrt(var + 1e-5)
            o_ref[h] = (c * inv * g + bt).reshape(bb, l, e)

    return pl.pallas_call(
        body,
        grid=(b2 // bb,),
        in_specs=[
            pl.BlockSpec((rb, 2 * e), lambda i: (i, 0)),
            pl.BlockSpec((rb, 2 * e), lambda i: (0, 0)),
            pl.BlockSpec((1, e), lambda i: (0, 0)),
            pl.BlockSpec((1, e), lambda i: (0, 0)),
        ],
        out_specs=pl.BlockSpec((2, bb, l, e), lambda i: (0, i, 0, 0)),
        out_shape=jax.ShapeDtypeStruct((2, b2, l, e), jnp.float32),
    )(xp, pos_tiled, gamma.reshape(1, e), beta.reshape(1, e))


def kernel(input_ids, token_type_ids, word_table, pos_table, type_table, ln_gamma, ln_beta):
    b, l = input_ids.shape
    e = word_table.shape[1]
    bb = 8
    ctable = (word_table[:, None, :] + type_table[None, :, :]).reshape(-1, e)
    cids = (input_ids * 2 + token_type_ids).reshape(b * l)
    xp = _sc_gather_packed(cids, ctable, chunk_rows=256)
    pos_tiled = jnp.tile(pos_table[:l], (bb, 2))
    out = _tc_add_ln(xp, pos_tiled, ln_gamma, ln_beta, b, l, e, bb)
    return out.reshape(b, l, e)


# speedup vs baseline: 1.4948x; 1.1573x over previous
import functools

import jax
import jax.numpy as jnp
from jax import lax
from jax.experimental import pallas as pl
from jax.experimental.pallas import tpu as pltpu
from jax.experimental.pallas import tpu_sc as plsc

_NC = 2
_NS = 16
_NW = _NC * _NS


def _sc_gather_packed(flat_ids, table, chunk_rows):
    n = flat_ids.shape[0]
    e = table.shape[1]
    n2 = n // 2
    per_w = n2 // _NW
    n_chunks = per_w // chunk_rows

    mesh = plsc.VectorSubcoreMesh(
        core_axis_name="c", subcore_axis_name="s", num_cores=_NC, num_subcores=_NS
    )

    @functools.partial(
        pl.kernel,
        out_type=jax.ShapeDtypeStruct((n2, 2 * e), jnp.float32),
        mesh=mesh,
        scratch_types=[
            pltpu.VMEM((chunk_rows,), jnp.int32),
            pltpu.VMEM((chunk_rows,), jnp.int32),
            pltpu.VMEM((chunk_rows, e), jnp.float32),
            pltpu.VMEM((chunk_rows, e), jnp.float32),
            pltpu.SemaphoreType.DMA,
        ],
        compiler_params=pltpu.CompilerParams(use_tc_tiling_on_sc=False),
    )
    def gather_kernel(ids_hbm, table_hbm, out_hbm, idx_l, idx_r, lv, rv, sem):
        wid = lax.axis_index("s") * _NC + lax.axis_index("c")
        base = wid * per_w

        def body(i, carry):
            off = pl.multiple_of(base + i * chunk_rows, 8)
            pltpu.sync_copy(ids_hbm.at[pl.ds(off, chunk_rows)], idx_l)
            pltpu.sync_copy(ids_hbm.at[pl.ds(n2 + off, chunk_rows)], idx_r)
            cl = pltpu.async_copy(table_hbm.at[idx_l], lv, sem)
            cr = pltpu.async_copy(table_hbm.at[idx_r], rv, sem)
            cl.wait()
            cr.wait()
            pltpu.sync_copy(lv, out_hbm.at[pl.ds(off, chunk_rows), pl.ds(0, e)])
            pltpu.sync_copy(rv, out_hbm.at[pl.ds(off, chunk_rows), pl.ds(e, e)])
            return carry

        lax.fori_loop(0, n_chunks, body, 0)

    return gather_kernel(flat_ids, table)


def _tc_add_ln(xp, pos_tiled, gamma, beta, b, l, e, bb):
    b2 = b // 2
    rb = bb * l

    def body(x_ref, pos_ref, g_ref, b_ref, o_ref):
        x = x_ref[...] + pos_ref[...]
        g = g_ref[...]
        bt = b_ref[...]
        for h in range(2):
            xh = x[:, h * e:(h + 1) * e]
            mean = jnp.mean(xh, axis=-1, keepdims=True)
            c = xh - mean
            var = jnp.mean(c * c, axis=-1, keepdims=True)
            inv = lax.rsqrt(var + 1e-5)
            o_ref[h] = (c * inv * g + bt).reshape(bb, l, e)

    return pl.pallas_call(
        body,
        grid=(b2 // bb,),
        in_specs=[
            pl.BlockSpec((rb, 2 * e), lambda i: (i, 0)),
            pl.BlockSpec((rb, 2 * e), lambda i: (0, 0)),
            pl.BlockSpec((1, e), lambda i: (0, 0)),
            pl.BlockSpec((1, e), lambda i: (0, 0)),
        ],
        out_specs=pl.BlockSpec((2, bb, l, e), lambda i: (0, i, 0, 0)),
        out_shape=jax.ShapeDtypeStruct((2, b2, l, e), jnp.float32),
    )(xp, pos_tiled, gamma.reshape(1, e), beta.reshape(1, e))


def kernel(input_ids, token_type_ids, word_table, pos_table, type_table, ln_gamma, ln_beta):
    b, l = input_ids.shape
    e = word_table.shape[1]
    bb = 8
    ctable = jnp.concatenate(
        [word_table + type_table[0], word_table + type_table[1]], axis=1
    ).reshape(-1, e)
    cids = (input_ids * 2 + token_type_ids).reshape(b * l)
    xp = _sc_gather_packed(cids, ctable, chunk_rows=256)
    pos_tiled = jnp.tile(pos_table[:l], (bb, 2))
    out = _tc_add_ln(xp, pos_tiled, ln_gamma, ln_beta, b, l, e, bb)
    return out.reshape(b, l, e)


# speedup vs baseline: 1.6228x; 1.0857x over previous
import functools

import jax
import jax.numpy as jnp
from jax import lax
from jax.experimental import pallas as pl
from jax.experimental.pallas import tpu as pltpu
from jax.experimental.pallas import tpu_sc as plsc

_NC = 2
_NS = 16
_NW = _NC * _NS


def _sc_gather_packed(flat_ids, table, chunk_rows):
    n = flat_ids.shape[0]
    e = table.shape[1]
    n2 = n // 2
    per_w = n2 // _NW
    n_chunks = per_w // chunk_rows

    mesh = plsc.VectorSubcoreMesh(
        core_axis_name="c", subcore_axis_name="s", num_cores=_NC, num_subcores=_NS
    )

    @functools.partial(
        pl.kernel,
        out_type=jax.ShapeDtypeStruct((n2, 2 * e), jnp.float32),
        mesh=mesh,
        scratch_types=[
            pltpu.VMEM((chunk_rows,), jnp.int32),
            pltpu.VMEM((chunk_rows,), jnp.int32),
            pltpu.VMEM((chunk_rows, e), jnp.float32),
            pltpu.VMEM((chunk_rows, e), jnp.float32),
            pltpu.SemaphoreType.DMA,
        ],
        compiler_params=pltpu.CompilerParams(use_tc_tiling_on_sc=False),
    )
    def gather_kernel(ids_hbm, table_hbm, out_hbm, idx_l, idx_r, lv, rv, sem):
        wid = lax.axis_index("s") * _NC + lax.axis_index("c")
        base = wid * per_w

        def body(i, carry):
            off = pl.multiple_of(base + i * chunk_rows, 8)
            pltpu.sync_copy(ids_hbm.at[pl.ds(off, chunk_rows)], idx_l)
            pltpu.sync_copy(ids_hbm.at[pl.ds(n2 + off, chunk_rows)], idx_r)
            cl = pltpu.async_copy(table_hbm.at[idx_l], lv, sem)
            cr = pltpu.async_copy(table_hbm.at[idx_r], rv, sem)
            cl.wait()
            cr.wait()
            pltpu.sync_copy(lv, out_hbm.at[pl.ds(off, chunk_rows), pl.ds(0, e)])
            pltpu.sync_copy(rv, out_hbm.at[pl.ds(off, chunk_rows), pl.ds(e, e)])
            return carry

        lax.fori_loop(0, n_chunks, body, 0)

    return gather_kernel(flat_ids, table)


def _tc_add_ln(xp, pos_tiled, gamma, beta, b, l, e, bb):
    b2 = b // 2
    rb = bb * l

    def body(x_ref, pos_ref, g_ref, b_ref, o_ref):
        x = x_ref[...] + pos_ref[...]
        g = g_ref[...]
        bt = b_ref[...]
        for h in range(2):
            xh = x[:, h * e:(h + 1) * e]
            mean = jnp.mean(xh, axis=-1, keepdims=True)
            c = xh - mean
            var = jnp.mean(c * c, axis=-1, keepdims=True)
            inv = lax.rsqrt(var + 1e-5)
            o_ref[h] = (c * inv * g + bt).reshape(bb, l, e)

    return pl.pallas_call(
        body,
        grid=(b2 // bb,),
        in_specs=[
            pl.BlockSpec((rb, 2 * e), lambda i: (i, 0)),
            pl.BlockSpec((rb, 2 * e), lambda i: (0, 0)),
            pl.BlockSpec((1, e), lambda i: (0, 0)),
            pl.BlockSpec((1, e), lambda i: (0, 0)),
        ],
        out_specs=pl.BlockSpec((2, bb, l, e), lambda i: (0, i, 0, 0)),
        out_shape=jax.ShapeDtypeStruct((2, b2, l, e), jnp.float32),
    )(xp, pos_tiled, gamma.reshape(1, e), beta.reshape(1, e))


def kernel(input_ids, token_type_ids, word_table, pos_table, type_table, ln_gamma, ln_beta):
    b, l = input_ids.shape
    e = word_table.shape[1]
    bb = 16
    ctable = jnp.concatenate(
        [word_table + type_table[0], word_table + type_table[1]], axis=1
    ).reshape(-1, e)
    cids = (input_ids * 2 + token_type_ids).reshape(b * l)
    xp = _sc_gather_packed(cids, ctable, chunk_rows=512)
    pos_tiled = jnp.tile(pos_table[:l], (bb, 2))
    out = _tc_add_ln(xp, pos_tiled, ln_gamma, ln_beta, b, l, e, bb)
    return out.reshape(b, l, e)


# speedup vs baseline: 1.6788x; 1.0345x over previous
import functools

import jax
import jax.numpy as jnp
from jax import lax
from jax.experimental import pallas as pl
from jax.experimental.pallas import tpu as pltpu
from jax.experimental.pallas import tpu_sc as plsc

_NC = 2
_NS = 16
_NW = _NC * _NS


def _sc_gather_packed(flat_ids, table, chunk_rows):
    n = flat_ids.shape[0]
    e = table.shape[1]
    n2 = n // 2
    per_w = n2 // _NW
    n_chunks = per_w // chunk_rows

    mesh = plsc.VectorSubcoreMesh(
        core_axis_name="c", subcore_axis_name="s", num_cores=_NC, num_subcores=_NS
    )

    @functools.partial(
        pl.kernel,
        out_type=jax.ShapeDtypeStruct((n2, 2 * e), jnp.float32),
        mesh=mesh,
        scratch_types=[
            pltpu.VMEM((chunk_rows,), jnp.int32),
            pltpu.VMEM((chunk_rows,), jnp.int32),
            pltpu.VMEM((chunk_rows, e), jnp.float32),
            pltpu.VMEM((chunk_rows, e), jnp.float32),
            pltpu.SemaphoreType.DMA,
        ],
        compiler_params=pltpu.CompilerParams(use_tc_tiling_on_sc=False),
    )
    def gather_kernel(ids_hbm, table_hbm, out_hbm, idx_l, idx_r, lv, rv, sem):
        wid = lax.axis_index("s") * _NC + lax.axis_index("c")
        base = wid * per_w

        def body(i, carry):
            off = pl.multiple_of(base + i * chunk_rows, 8)
            pltpu.sync_copy(ids_hbm.at[pl.ds(off, chunk_rows)], idx_l)
            pltpu.sync_copy(ids_hbm.at[pl.ds(n2 + off, chunk_rows)], idx_r)
            cl = pltpu.async_copy(table_hbm.at[idx_l], lv, sem)
            cr = pltpu.async_copy(table_hbm.at[idx_r], rv, sem)
            cl.wait()
            cr.wait()
            pltpu.sync_copy(lv, out_hbm.at[pl.ds(off, chunk_rows), pl.ds(0, e)])
            pltpu.sync_copy(rv, out_hbm.at[pl.ds(off, chunk_rows), pl.ds(e, e)])
            return carry

        lax.fori_loop(0, n_chunks, body, 0)

    return gather_kernel(flat_ids, table)


def _tc_add_ln(xp, pos_tiled, gamma, beta, b, l, e, bb):
    b2 = b // 2
    rb = bb * l

    def body(x_ref, pos_ref, g_ref, b_ref, o_ref):
        x = x_ref[...] + pos_ref[...]
        g = g_ref[...]
        bt = b_ref[...]
        for h in range(2):
            xh = x[:, h * e:(h + 1) * e]
            mean = jnp.mean(xh, axis=-1, keepdims=True)
            c = xh - mean
            var = jnp.mean(c * c, axis=-1, keepdims=True)
            inv = lax.rsqrt(var + 1e-5)
            o_ref[h] = (c * inv * g + bt).reshape(bb, l, e)

    return pl.pallas_call(
        body,
        grid=(b2 // bb,),
        in_specs=[
            pl.BlockSpec((rb, 2 * e), lambda i: (i, 0)),
            pl.BlockSpec((rb, 2 * e), lambda i: (0, 0)),
            pl.BlockSpec((1, e), lambda i: (0, 0)),
            pl.BlockSpec((1, e), lambda i: (0, 0)),
        ],
        out_specs=pl.BlockSpec((2, bb, l, e), lambda i: (0, i, 0, 0)),
        out_shape=jax.ShapeDtypeStruct((2, b2, l, e), jnp.float32),
    )(xp, pos_tiled, gamma.reshape(1, e), beta.reshape(1, e))


def kernel(input_ids, token_type_ids, word_table, pos_table, type_table, ln_gamma, ln_beta):
    b, l = input_ids.shape
    e = word_table.shape[1]
    bb = 32
    ctable = jnp.concatenate(
        [word_table + type_table[0], word_table + type_table[1]], axis=1
    ).reshape(-1, e)
    cids = (input_ids * 2 + token_type_ids).reshape(b * l)
    xp = _sc_gather_packed(cids, ctable, chunk_rows=800)
    pos_tiled = jnp.tile(pos_table[:l], (bb, 2))
    out = _tc_add_ln(xp, pos_tiled, ln_gamma, ln_beta, b, l, e, bb)
    return out.reshape(b, l, e)
